# baseline (speedup 1.0000x reference)
import functools

import jax
import jax.numpy as jnp
from jax import lax
from jax.experimental import pallas as pl
from jax.experimental.pallas import tpu as pltpu
from jax.experimental.pallas import tpu_sc as plsc

TOP_K = 8
NUM_EXPERTS = 64
HIDDEN = 2048
TOKENS = 8192

NC = 2
NS = 16
LANES = 16
NW = NC * NS
ROWS_PER_W = TOKENS // NW


def _matmul_body(x_ref, w_ref, b_ref, o_ref):
    x = x_ref[...]
    w = w_ref[...]
    kc = 512
    acc = None
    for k0 in range(0, HIDDEN, kc):
        d = lax.dot_general(
            x[:, k0:k0 + kc], w[:, k0:k0 + kc],
            dimension_numbers=(((1,), (1,)), ((), ())),
            preferred_element_type=jnp.float32,
        )
        acc = d if acc is None else acc + d
    o_ref[...] = acc + b_ref[...]


def _router_logits(hidden_states, weight, bias):
    bt = 1024
    return pl.pallas_call(
        _matmul_body,
        grid=(TOKENS // bt,),
        in_specs=[
            pl.BlockSpec((bt, HIDDEN), lambda i: (i, 0)),
            pl.BlockSpec((NUM_EXPERTS, HIDDEN), lambda i: (0, 0)),
            pl.BlockSpec((1, NUM_EXPERTS), lambda i: (0, 0)),
        ],
        out_specs=pl.BlockSpec((bt, NUM_EXPERTS), lambda i: (i, 0)),
        out_shape=jax.ShapeDtypeStruct((TOKENS, NUM_EXPERTS), jnp.float32),
    )(hidden_states, weight, bias.reshape(1, NUM_EXPERTS))


def _merge_sorted(ka, va, kb, vb):
    krb = lax.rev(kb, (0,))
    vrb = lax.rev(vb, (0,))
    cond = ka >= krb
    mk = jnp.where(cond, ka, krb)
    mv = jnp.where(cond, va, vrb)
    return plsc.sort_key_val(mk, mv, descending=True)


def _route_body(lg_hbm, sc_hbm, ix_hbm, lg_v, sc_v, ix_v):
    wid = lax.axis_index("s") * NC + lax.axis_index("c")
    row0 = wid * ROWS_PER_W
    pltpu.sync_copy(lg_hbm.at[pl.ds(row0, ROWS_PER_W)], lg_v)

    lane = lax.iota(jnp.int32, LANES)
    m8 = lane < TOP_K

    @plsc.parallel_loop(0, ROWS_PER_W, unroll=4)
    def _row(row):
        ks, vs = [], []
        for c in range(4):
            k = lg_v[row, pl.ds(LANES * c, LANES)]
            sk, sv = plsc.sort_key_val(k, lane + LANES * c, descending=True)
            ks.append(sk)
            vs.append(sv)
        k01, v01 = _merge_sorted(ks[0], vs[0], ks[1], vs[1])
        k23, v23 = _merge_sorted(ks[2], vs[2], ks[3], vs[3])
        kf, vf = _merge_sorted(k01, v01, k23, v23)

        e = jnp.exp(kf - jnp.max(kf))
        ez = jnp.where(m8, e, 0.0)
        p = ez / jnp.sum(ez)

        for c in range(4):
            sc_v[row, pl.ds(LANES * c, LANES)] = jnp.zeros((LANES,), jnp.float32)
        rvec = jnp.broadcast_to(row, (LANES,))
        plsc.store_scatter(sc_v, [rvec, vf], p, mask=m8)
        plsc.store_scatter(ix_v, [rvec, lane], vf, mask=m8)

    pltpu.sync_copy(sc_v, sc_hbm.at[pl.ds(row0, ROWS_PER_W)])
    pltpu.sync_copy(ix_v, ix_hbm.at[pl.ds(row0, ROWS_PER_W)])


@functools.partial(
    pl.kernel,
    out_type=(
        jax.ShapeDtypeStruct((TOKENS, NUM_EXPERTS), jnp.float32),
        jax.ShapeDtypeStruct((TOKENS, TOP_K), jnp.int32),
    ),
    mesh=plsc.VectorSubcoreMesh(core_axis_name="c", subcore_axis_name="s"),
    scratch_types=[
        pltpu.VMEM((ROWS_PER_W, NUM_EXPERTS), jnp.float32),
        pltpu.VMEM((ROWS_PER_W, NUM_EXPERTS), jnp.float32),
        pltpu.VMEM((ROWS_PER_W, TOP_K), jnp.int32),
    ],
    compiler_params=pltpu.CompilerParams(
        needs_layout_passes=False,
        skip_device_barrier=True,
        disable_bounds_checks=True,
        disable_semaphore_checks=True,
    ),
)
def _route(lg_hbm, sc_hbm, ix_hbm, lg_v, sc_v, ix_v):
    _route_body(lg_hbm, sc_hbm, ix_hbm, lg_v, sc_v, ix_v)


def kernel(hidden_states, weight, bias):
    logits = _router_logits(hidden_states, weight, bias)
    scores, indices = _route(logits)
    return (scores, indices)


# speedup vs baseline: 1.0048x; 1.0048x over previous
import functools

import jax
import jax.numpy as jnp
from jax import lax
from jax.experimental import pallas as pl
from jax.experimental.pallas import tpu as pltpu
from jax.experimental.pallas import tpu_sc as plsc

TOP_K = 8
NUM_EXPERTS = 64
HIDDEN = 2048
TOKENS = 8192

NC = 2
NS = 16
LANES = 16
NW = NC * NS
ROWS_PER_W = TOKENS // NW


def _matmul_body(x_ref, w_ref, b_ref, o_ref):
    x = x_ref[...]
    w = w_ref[...]
    kc = 512
    acc = None
    for k0 in range(0, HIDDEN, kc):
        d = lax.dot_general(
            x[:, k0:k0 + kc], w[:, k0:k0 + kc],
            dimension_numbers=(((1,), (1,)), ((), ())),
            preferred_element_type=jnp.float32,
        )
        acc = d if acc is None else acc + d
    o_ref[...] = acc + b_ref[...]


def _router_logits(hidden_states, weight, bias):
    bt = 1024
    return pl.pallas_call(
        _matmul_body,
        grid=(TOKENS // bt,),
        in_specs=[
            pl.BlockSpec((bt, HIDDEN), lambda i: (i, 0)),
            pl.BlockSpec((NUM_EXPERTS, HIDDEN), lambda i: (0, 0)),
            pl.BlockSpec((1, NUM_EXPERTS), lambda i: (0, 0)),
        ],
        out_specs=pl.BlockSpec((bt, NUM_EXPERTS), lambda i: (i, 0)),
        out_shape=jax.ShapeDtypeStruct((TOKENS, NUM_EXPERTS), jnp.float32),
    )(hidden_states, weight, bias.reshape(1, NUM_EXPERTS))


def _merge_sorted(ka, va, kb, vb):
    krb = lax.rev(kb, (0,))
    vrb = lax.rev(vb, (0,))
    cond = ka >= krb
    mk = jnp.where(cond, ka, krb)
    mv = jnp.where(cond, va, vrb)
    return plsc.sort_key_val(mk, mv, descending=True)


N_CHUNK = 4
CHUNK = ROWS_PER_W // N_CHUNK


def _route_body(lg_hbm, sc_hbm, ix_hbm, lg_v, sc_v, ix_v,
                sem_in0, sem_in1, sem_in2, sem_in3, sem_out):
    wid = lax.axis_index("s") * NC + lax.axis_index("c")
    row0 = wid * ROWS_PER_W
    sems_in = (sem_in0, sem_in1, sem_in2, sem_in3)

    ins = []
    for c in range(N_CHUNK):
        ins.append(pltpu.async_copy(
            lg_hbm.at[pl.ds(row0 + c * CHUNK, CHUNK)],
            lg_v.at[pl.ds(c * CHUNK, CHUNK)],
            sems_in[c],
        ))

    lane = lax.iota(jnp.int32, LANES)
    m8 = lane < TOP_K
    outs = []
    for c in range(N_CHUNK):
        ins[c].wait()

        @plsc.parallel_loop(c * CHUNK, (c + 1) * CHUNK, unroll=4)
        def _row(row):
            ks, vs = [], []
            for q in range(4):
                k = lg_v[row, pl.ds(LANES * q, LANES)]
                sk, sv = plsc.sort_key_val(k, lane + LANES * q, descending=True)
                ks.append(sk)
                vs.append(sv)
            k01, v01 = _merge_sorted(ks[0], vs[0], ks[1], vs[1])
            k23, v23 = _merge_sorted(ks[2], vs[2], ks[3], vs[3])
            kf, vf = _merge_sorted(k01, v01, k23, v23)

            e = jnp.exp(kf - jnp.max(kf))
            ez = jnp.where(m8, e, 0.0)
            p = ez / jnp.sum(ez)

            for q in range(4):
                sc_v[row, pl.ds(LANES * q, LANES)] = jnp.zeros((LANES,), jnp.float32)
            rvec = jnp.broadcast_to(row, (LANES,))
            plsc.store_scatter(sc_v, [rvec, vf], p, mask=m8)
            plsc.store_scatter(ix_v, [rvec, lane], vf, mask=m8)

        outs.append(pltpu.async_copy(
            sc_v.at[pl.ds(c * CHUNK, CHUNK)],
            sc_hbm.at[pl.ds(row0 + c * CHUNK, CHUNK)],
            sem_out,
        ))
        outs.append(pltpu.async_copy(
            ix_v.at[pl.ds(c * CHUNK, CHUNK)],
            ix_hbm.at[pl.ds(row0 + c * CHUNK, CHUNK)],
            sem_out,
        ))
    for o in outs:
        o.wait()


@functools.partial(
    pl.kernel,
    out_type=(
        jax.ShapeDtypeStruct((TOKENS, NUM_EXPERTS), jnp.float32),
        jax.ShapeDtypeStruct((TOKENS, TOP_K), jnp.int32),
    ),
    mesh=plsc.VectorSubcoreMesh(core_axis_name="c", subcore_axis_name="s"),
    scratch_types=[
        pltpu.VMEM((ROWS_PER_W, NUM_EXPERTS), jnp.float32),
        pltpu.VMEM((ROWS_PER_W, NUM_EXPERTS), jnp.float32),
        pltpu.VMEM((ROWS_PER_W, TOP_K), jnp.int32),
        pltpu.SemaphoreType.DMA,
        pltpu.SemaphoreType.DMA,
        pltpu.SemaphoreType.DMA,
        pltpu.SemaphoreType.DMA,
        pltpu.SemaphoreType.DMA,
    ],
    compiler_params=pltpu.CompilerParams(needs_layout_passes=False),
)
def _route(lg_hbm, sc_hbm, ix_hbm, lg_v, sc_v, ix_v, s0, s1, s2, s3, so):
    _route_body(lg_hbm, sc_hbm, ix_hbm, lg_v, sc_v, ix_v, s0, s1, s2, s3, so)


def kernel(hidden_states, weight, bias):
    logits = _router_logits(hidden_states, weight, bias)
    scores, indices = _route(logits)
    return (scores, indices)
